# baseline (speedup 1.0000x reference)
import functools

import jax
import jax.numpy as jnp
from jax import lax
from jax.experimental import pallas as pl
from jax.experimental.pallas import tpu as pltpu
from jax.experimental.pallas import tpu_sc as plsc

TOTAL_TOKENS = 8192
HIDDEN = 2048
INTER = 5632
B_EXPERT = 512

BLK_I = 256
N_BLK = INTER // BLK_I

_NC, _NS = 2, 16
_NW = _NC * _NS
_B_PER_W = B_EXPERT // _NW


def _gather_body(x_hbm, idx_hbm, out_hbm, idx_v, rows_v, sem):
    wid = lax.axis_index("s") * _NC + lax.axis_index("c")
    base = wid * _B_PER_W
    pltpu.sync_copy(idx_hbm.at[pl.ds(base, _B_PER_W)], idx_v)
    pltpu.async_copy(x_hbm.at[idx_v], rows_v, sem).wait()
    pltpu.sync_copy(rows_v, out_hbm.at[pl.ds(base, _B_PER_W)])


@functools.cache
def _sc_gather():
    return pl.kernel(
        _gather_body,
        out_type=jax.ShapeDtypeStruct((B_EXPERT, HIDDEN), jnp.float32),
        mesh=plsc.VectorSubcoreMesh(core_axis_name="c", subcore_axis_name="s"),
        scratch_types=[
            pltpu.VMEM((_B_PER_W,), jnp.int32),
            pltpu.VMEM((_B_PER_W, HIDDEN), jnp.float32),
            pltpu.SemaphoreType.DMA,
        ],
    )


_NT = (((1,), (1,)), ((), ()))


def _ffn_body(xs_ref, wg_ref, wu_ref, wd_ref, w_ref, out_ref, acc_ref):
    i = pl.program_id(0)
    xb = xs_ref[...]
    gab = lax.dot_general(xb, wg_ref[...].reshape(2 * BLK_I, HIDDEN), _NT,
                          preferred_element_type=jnp.float32)
    ga = gab[:, :BLK_I]
    gb = gab[:, BLK_I:]
    up = lax.dot_general(xb, wu_ref[...], _NT,
                         preferred_element_type=jnp.float32)
    h = ga * (1.0 / (1.0 + jnp.exp(-gb))) * up
    contrib = lax.dot_general(h, wd_ref[...], _NT,
                              preferred_element_type=jnp.float32)

    @pl.when(i == 0)
    def _init():
        acc_ref[...] = contrib

    @pl.when(i > 0)
    def _accum():
        acc_ref[...] += contrib

    @pl.when(i == N_BLK - 1)
    def _scale():
        out_ref[...] = acc_ref[...] * w_ref[...]


def _tc_ffn(xs, weight, W_gate, W_up, W_down):
    return pl.pallas_call(
        _ffn_body,
        grid=(N_BLK,),
        in_specs=[
            pl.BlockSpec((B_EXPERT, HIDDEN), lambda i: (0, 0)),
            pl.BlockSpec((2, BLK_I, HIDDEN), lambda i: (0, i, 0)),
            pl.BlockSpec((BLK_I, HIDDEN), lambda i: (i, 0)),
            pl.BlockSpec((HIDDEN, BLK_I), lambda i: (0, i)),
            pl.BlockSpec((B_EXPERT, 1), lambda i: (0, 0)),
        ],
        out_specs=pl.BlockSpec((B_EXPERT, HIDDEN), lambda i: (0, 0)),
        out_shape=jax.ShapeDtypeStruct((B_EXPERT, HIDDEN), jnp.float32),
        scratch_shapes=[pltpu.VMEM((B_EXPERT, HIDDEN), jnp.float32)],
        compiler_params=pltpu.CompilerParams(
            dimension_semantics=("arbitrary",),
        ),
    )(xs, W_gate.reshape(2, INTER, HIDDEN), W_up, W_down, weight)


def kernel(x, top_x, weight, W_gate, W_up, W_down):
    xs = _sc_gather()(x, top_x.astype(jnp.int32))
    return _tc_ffn(xs, weight, W_gate, W_up, W_down)


# speedup vs baseline: 1.2470x; 1.2470x over previous
import functools

import jax
import jax.numpy as jnp
from jax import lax
from jax.experimental import pallas as pl
from jax.experimental.pallas import tpu as pltpu
from jax.experimental.pallas import tpu_sc as plsc

TOTAL_TOKENS = 8192
HIDDEN = 2048
INTER = 5632
B_EXPERT = 512

BLK_I = 512
N_BLK = INTER // BLK_I

_NC, _NS = 2, 16
_NW = _NC * _NS
_B_PER_W = B_EXPERT // _NW


def _gather_body(x_hbm, idx_hbm, out_hbm, idx_v, rows_v, sem):
    wid = lax.axis_index("s") * _NC + lax.axis_index("c")
    base = wid * _B_PER_W
    pltpu.sync_copy(idx_hbm.at[pl.ds(base, _B_PER_W)], idx_v)
    pltpu.async_copy(x_hbm.at[idx_v], rows_v, sem).wait()
    pltpu.sync_copy(rows_v, out_hbm.at[pl.ds(base, _B_PER_W)])


@functools.cache
def _sc_gather():
    return pl.kernel(
        _gather_body,
        out_type=jax.ShapeDtypeStruct((B_EXPERT, HIDDEN), jnp.float32),
        mesh=plsc.VectorSubcoreMesh(core_axis_name="c", subcore_axis_name="s"),
        scratch_types=[
            pltpu.VMEM((_B_PER_W,), jnp.int32),
            pltpu.VMEM((_B_PER_W, HIDDEN), jnp.float32),
            pltpu.SemaphoreType.DMA,
        ],
    )


_NT = (((1,), (1,)), ((), ()))


def _ffn_body(xs_ref, wg_ref, wu_ref, wd_ref, w_ref, out_ref, acc_ref, h_ref):
    i = pl.program_id(0)
    h_prev = h_ref[...]

    @pl.when(i > 0)
    def _down():
        contrib = lax.dot_general(h_prev, wd_ref[...], _NT,
                                  preferred_element_type=jnp.float32)

        @pl.when(i == 1)
        def _init():
            acc_ref[...] = contrib

        @pl.when(i > 1)
        def _accum():
            acc_ref[...] += contrib

    @pl.when(i < N_BLK)
    def _gate_up():
        xb = xs_ref[...]
        gab = lax.dot_general(xb, wg_ref[...].reshape(2 * BLK_I, HIDDEN), _NT,
                              preferred_element_type=jnp.float32)
        ga = gab[:, :BLK_I]
        gb = gab[:, BLK_I:]
        up = lax.dot_general(xb, wu_ref[...], _NT,
                             preferred_element_type=jnp.float32)
        h_ref[...] = ga * (1.0 / (1.0 + jnp.exp(-gb))) * up

    @pl.when(i == N_BLK)
    def _scale():
        out_ref[...] = acc_ref[...] * w_ref[...]


def _tc_ffn(xs, weight, W_gate, W_up, W_down):
    return pl.pallas_call(
        _ffn_body,
        grid=(N_BLK + 1,),
        in_specs=[
            pl.BlockSpec((B_EXPERT, HIDDEN), lambda i: (0, 0)),
            pl.BlockSpec((2, BLK_I, HIDDEN),
                         lambda i: (0, jnp.minimum(i, N_BLK - 1), 0)),
            pl.BlockSpec((BLK_I, HIDDEN),
                         lambda i: (jnp.minimum(i, N_BLK - 1), 0)),
            pl.BlockSpec((HIDDEN, BLK_I),
                         lambda i: (0, jnp.maximum(i - 1, 0))),
            pl.BlockSpec((B_EXPERT, 1), lambda i: (0, 0)),
        ],
        out_specs=pl.BlockSpec((B_EXPERT, HIDDEN), lambda i: (0, 0)),
        out_shape=jax.ShapeDtypeStruct((B_EXPERT, HIDDEN), jnp.float32),
        scratch_shapes=[pltpu.VMEM((B_EXPERT, HIDDEN), jnp.float32),
                        pltpu.VMEM((B_EXPERT, BLK_I), jnp.float32)],
        compiler_params=pltpu.CompilerParams(
            dimension_semantics=("arbitrary",),
        ),
    )(xs, W_gate.reshape(2, INTER, HIDDEN), W_up, W_down, weight)


def kernel(x, top_x, weight, W_gate, W_up, W_down):
    xs = _sc_gather()(x, top_x.astype(jnp.int32))
    return _tc_ffn(xs, weight, W_gate, W_up, W_down)


# speedup vs baseline: 1.2662x; 1.0154x over previous
import functools

import jax
import jax.numpy as jnp
from jax import lax
from jax.experimental import pallas as pl
from jax.experimental.pallas import tpu as pltpu
from jax.experimental.pallas import tpu_sc as plsc

TOTAL_TOKENS = 8192
HIDDEN = 2048
INTER = 5632
B_EXPERT = 512

BLK_I = 512
N_BLK = INTER // BLK_I

_NC, _NS = 1, 16
_NW = _NC * _NS
_B_PER_W = B_EXPERT // _NW


def _gather_body(x_hbm, idx_hbm, out_hbm, idx_v, rows_v, sem):
    wid = lax.axis_index("s") * _NC + lax.axis_index("c")
    base = wid * _B_PER_W
    pltpu.sync_copy(idx_hbm.at[pl.ds(base, _B_PER_W)], idx_v)
    pltpu.async_copy(x_hbm.at[idx_v], rows_v, sem).wait()
    pltpu.sync_copy(rows_v, out_hbm.at[pl.ds(base, _B_PER_W)])


@functools.cache
def _sc_gather():
    return pl.kernel(
        _gather_body,
        out_type=jax.ShapeDtypeStruct((B_EXPERT, HIDDEN), jnp.float32),
        mesh=plsc.VectorSubcoreMesh(core_axis_name="c", subcore_axis_name="s", num_cores=1),
        scratch_types=[
            pltpu.VMEM((_B_PER_W,), jnp.int32),
            pltpu.VMEM((_B_PER_W, HIDDEN), jnp.float32),
            pltpu.SemaphoreType.DMA,
        ],
    )


_NT = (((1,), (1,)), ((), ()))


def _ffn_body(xs_ref, wg_ref, wu_ref, wd_ref, w_ref, out_ref, acc_ref):
    i = pl.program_id(0)
    xb = xs_ref[...]
    gab = lax.dot_general(xb, wg_ref[...].reshape(2 * BLK_I, HIDDEN), _NT,
                          preferred_element_type=jnp.float32)
    ga = gab[:, :BLK_I]
    gb = gab[:, BLK_I:]
    up = lax.dot_general(xb, wu_ref[...], _NT,
                         preferred_element_type=jnp.float32)
    h = ga * (1.0 / (1.0 + jnp.exp(-gb))) * up
    contrib = lax.dot_general(h, wd_ref[...], _NT,
                              preferred_element_type=jnp.float32)

    @pl.when(i == 0)
    def _init():
        acc_ref[...] = contrib

    @pl.when(i > 0)
    def _accum():
        acc_ref[...] += contrib

    @pl.when(i == N_BLK - 1)
    def _scale():
        out_ref[...] = acc_ref[...] * w_ref[...]


def _tc_ffn(xs, weight, W_gate, W_up, W_down):
    return pl.pallas_call(
        _ffn_body,
        grid=(N_BLK,),
        in_specs=[
            pl.BlockSpec((B_EXPERT, HIDDEN), lambda i: (0, 0)),
            pl.BlockSpec((2, BLK_I, HIDDEN), lambda i: (0, i, 0)),
            pl.BlockSpec((BLK_I, HIDDEN), lambda i: (i, 0)),
            pl.BlockSpec((HIDDEN, BLK_I), lambda i: (0, i)),
            pl.BlockSpec((B_EXPERT, 1), lambda i: (0, 0)),
        ],
        out_specs=pl.BlockSpec((B_EXPERT, HIDDEN), lambda i: (0, 0)),
        out_shape=jax.ShapeDtypeStruct((B_EXPERT, HIDDEN), jnp.float32),
        scratch_shapes=[pltpu.VMEM((B_EXPERT, HIDDEN), jnp.float32)],
        compiler_params=pltpu.CompilerParams(
            dimension_semantics=("arbitrary",),
        ),
    )(xs, W_gate.reshape(2, INTER, HIDDEN), W_up, W_down, weight)


def kernel(x, top_x, weight, W_gate, W_up, W_down):
    xs = _sc_gather()(x, top_x.astype(jnp.int32))
    return _tc_ffn(xs, weight, W_gate, W_up, W_down)


# speedup vs baseline: 1.2714x; 1.0041x over previous
import functools

import jax
import jax.numpy as jnp
from jax import lax
from jax.experimental import pallas as pl
from jax.experimental.pallas import tpu as pltpu
from jax.experimental.pallas import tpu_sc as plsc

TOTAL_TOKENS = 8192
HIDDEN = 2048
INTER = 5632
B_EXPERT = 512

BLK_I = 512
N_BLK = INTER // BLK_I

_NC, _NS = 2, 16
_NW = _NC * _NS
_B_PER_W = B_EXPERT // _NW


_HALF = _B_PER_W // 2


def _gather_body(x_hbm, idx_hbm, out_hbm, idx_a, idx_b, rows_a, rows_b,
                 sem_a, sem_b):
    wid = lax.axis_index("s") * _NC + lax.axis_index("c")
    base = wid * _B_PER_W
    pltpu.sync_copy(idx_hbm.at[pl.ds(base, _HALF)], idx_a)
    pltpu.sync_copy(idx_hbm.at[pl.ds(base + _HALF, _HALF)], idx_b)
    cp_a = pltpu.async_copy(x_hbm.at[idx_a], rows_a, sem_a)
    cp_b = pltpu.async_copy(x_hbm.at[idx_b], rows_b, sem_b)
    cp_a.wait()
    pltpu.sync_copy(rows_a, out_hbm.at[pl.ds(base, _HALF)])
    cp_b.wait()
    pltpu.sync_copy(rows_b, out_hbm.at[pl.ds(base + _HALF, _HALF)])


@functools.cache
def _sc_gather():
    return pl.kernel(
        _gather_body,
        out_type=jax.ShapeDtypeStruct((B_EXPERT, HIDDEN), jnp.float32),
        mesh=plsc.VectorSubcoreMesh(core_axis_name="c", subcore_axis_name="s"),
        scratch_types=[
            pltpu.VMEM((_HALF,), jnp.int32),
            pltpu.VMEM((_HALF,), jnp.int32),
            pltpu.VMEM((_HALF, HIDDEN), jnp.float32),
            pltpu.VMEM((_HALF, HIDDEN), jnp.float32),
            pltpu.SemaphoreType.DMA,
            pltpu.SemaphoreType.DMA,
        ],
    )


_NT = (((1,), (1,)), ((), ()))


def _ffn_body(xs_ref, wg_ref, wu_ref, wd_ref, w_ref, out_ref, acc_ref):
    i = pl.program_id(0)
    xb = xs_ref[...]
    gab = lax.dot_general(xb, wg_ref[...].reshape(2 * BLK_I, HIDDEN), _NT,
                          preferred_element_type=jnp.float32)
    ga = gab[:, :BLK_I]
    gb = gab[:, BLK_I:]
    up = lax.dot_general(xb, wu_ref[...], _NT,
                         preferred_element_type=jnp.float32)
    h = ga * (1.0 / (1.0 + jnp.exp(-gb))) * up
    contrib = lax.dot_general(h, wd_ref[...], _NT,
                              preferred_element_type=jnp.float32)

    @pl.when(i == 0)
    def _init():
        acc_ref[...] = contrib

    @pl.when(i > 0)
    def _accum():
        acc_ref[...] += contrib

    @pl.when(i == N_BLK - 1)
    def _scale():
        out_ref[...] = acc_ref[...] * w_ref[...]


def _tc_ffn(xs, weight, W_gate, W_up, W_down):
    return pl.pallas_call(
        _ffn_body,
        grid=(N_BLK,),
        in_specs=[
            pl.BlockSpec((B_EXPERT, HIDDEN), lambda i: (0, 0)),
            pl.BlockSpec((2, BLK_I, HIDDEN), lambda i: (0, i, 0)),
            pl.BlockSpec((BLK_I, HIDDEN), lambda i: (i, 0)),
            pl.BlockSpec((HIDDEN, BLK_I), lambda i: (0, i)),
            pl.BlockSpec((B_EXPERT, 1), lambda i: (0, 0)),
        ],
        out_specs=pl.BlockSpec((B_EXPERT, HIDDEN), lambda i: (0, 0)),
        out_shape=jax.ShapeDtypeStruct((B_EXPERT, HIDDEN), jnp.float32),
        scratch_shapes=[pltpu.VMEM((B_EXPERT, HIDDEN), jnp.float32)],
        compiler_params=pltpu.CompilerParams(
            dimension_semantics=("arbitrary",),
        ),
    )(xs, W_gate.reshape(2, INTER, HIDDEN), W_up, W_down, weight)


def kernel(x, top_x, weight, W_gate, W_up, W_down):
    xs = _sc_gather()(x, top_x.astype(jnp.int32))
    return _tc_ffn(xs, weight, W_gate, W_up, W_down)
